# Initial kernel scaffold; baseline (speedup 1.0000x reference)
import jax
import jax.numpy as jnp
from jax.experimental import pallas as pl


def kernel(vecs, v, rotations):
    raise NotImplementedError("write your pallas kernel here")


# speedup vs baseline: 3.4819x; 3.4819x over previous
import functools

import jax
import jax.numpy as jnp
from jax import lax
from jax.experimental import pallas as pl
from jax.experimental.pallas import tpu as pltpu
from jax.experimental.pallas import tpu_sc as plsc

_N_BUCKETS = 128
_N_HASHES = 4
_SELF_VAL = -50000.0
_CHUNK = 64
_CB = 8
_RBLK = 128


def _rank_body(vecs_ref, rot_ref, rank_ref, stf_ref, stl_ref, o_scr):
    p = pl.program_id(0)
    seqlen = vecs_ref.shape[1]
    nb = seqlen // _RBLK
    rot = rot_ref[0]

    iota_c = lax.broadcasted_iota(jnp.int32, (_RBLK, _N_BUCKETS), 1)
    lt = (lax.broadcasted_iota(jnp.int32, (_RBLK, _RBLK), 0)
          > lax.broadcasted_iota(jnp.int32, (_RBLK, _RBLK), 1)).astype(jnp.float32)

    wr_list = []
    hb_list = []
    for i in range(nb):
        blk = vecs_ref[0, pl.ds(i * _RBLK, _RBLK), :]
        d = jnp.dot(blk, rot, preferred_element_type=jnp.float32)
        full = jnp.concatenate([d, -d], axis=1)
        m = jnp.max(full, axis=1, keepdims=True)
        cand = jnp.where(full == m, iota_c, _N_BUCKETS)
        cstar = jnp.min(cand, axis=1, keepdims=True)
        onehot = (iota_c == cstar).astype(jnp.float32)
        o_scr[pl.ds(i * _RBLK, _RBLK), :] = onehot
        c = jnp.dot(lt, onehot, preferred_element_type=jnp.float32)
        wr_list.append(jnp.sum(c * onehot, axis=1))
        hb_list.append(jnp.sum(onehot, axis=0, keepdims=True))

    hb = jnp.concatenate(hb_list, axis=0)
    ltb = (lax.broadcasted_iota(jnp.int32, (nb, nb), 0)
           > lax.broadcasted_iota(jnp.int32, (nb, nb), 1)).astype(jnp.float32)
    boff = jnp.dot(ltb, hb, preferred_element_type=jnp.float32)
    htot = jnp.sum(hb, axis=0, keepdims=True)
    su = (lax.broadcasted_iota(jnp.int32, (_N_BUCKETS, _N_BUCKETS), 0)
          < lax.broadcasted_iota(jnp.int32, (_N_BUCKETS, _N_BUCKETS), 1)
          ).astype(jnp.float32)
    off = jnp.dot(htot, su, preferred_element_type=jnp.float32)
    base = off + boff

    gbase = (p * seqlen).astype(jnp.float32)
    iota64r = lax.broadcasted_iota(jnp.int32, (1, _CHUNK), 1).astype(jnp.float32)
    stf_acc = jnp.zeros((_CHUNK, 1), jnp.float32)
    stl_acc = jnp.zeros((1, _CHUNK), jnp.float32)
    for i in range(nb):
        onehot = o_scr[pl.ds(i * _RBLK, _RBLK), :]
        r = wr_list[i] + jnp.sum(onehot * base[i:i + 1, :], axis=1)
        rank_ref[0, 0, pl.ds(i * _RBLK, _RBLK)] = r + gbase
        rcol = r[:, None]
        tok_col = (lax.broadcasted_iota(jnp.int32, (_RBLK, 1), 0)
                   .astype(jnp.float32) + (i * _RBLK))
        m1 = (rcol == iota64r).astype(jnp.float32)
        m2 = (rcol == iota64r + (seqlen - _CHUNK)).astype(jnp.float32)
        stf_acc = stf_acc + lax.dot_general(
            m1, tok_col, (((0,), (0,)), ((), ())),
            preferred_element_type=jnp.float32)
        stl_acc = stl_acc + lax.dot_general(
            tok_col, m2, (((0,), (0,)), ((), ())),
            preferred_element_type=jnp.float32)
    stf_ref[0] = stf_acc
    stl_ref[0] = stl_acc


def _hash_rank(vecs, rot_t):
    b, seqlen, dim = vecs.shape
    nprog = b * _N_HASHES
    rank, stf, stl = pl.pallas_call(
        _rank_body,
        grid=(nprog,),
        in_specs=[
            pl.BlockSpec((1, seqlen, dim), lambda p: (p // _N_HASHES, 0, 0)),
            pl.BlockSpec((1, dim, dim), lambda p: (p % _N_HASHES, 0, 0)),
        ],
        out_specs=[
            pl.BlockSpec((1, 1, seqlen), lambda p: (p, 0, 0)),
            pl.BlockSpec((1, _CHUNK, 1), lambda p: (p, 0, 0)),
            pl.BlockSpec((1, 1, _CHUNK), lambda p: (p, 0, 0)),
        ],
        out_shape=[
            jax.ShapeDtypeStruct((nprog, 1, seqlen), jnp.float32),
            jax.ShapeDtypeStruct((nprog, _CHUNK, 1), jnp.float32),
            jax.ShapeDtypeStruct((nprog, 1, _CHUNK), jnp.float32),
        ],
        scratch_shapes=[pltpu.VMEM((seqlen, _N_BUCKETS), jnp.float32)],
    )(vecs, rot_t)
    return rank.reshape(nprog, seqlen), stf, stl


def _sc_scatter(qv_flat, grank):
    nseg, seqlen = grank.shape
    tot = nseg * seqlen
    width = qv_flat.shape[1]
    info = plsc.get_sparse_core_info()
    nw = info.num_cores * info.num_subcores
    tps = nw // nseg
    half = seqlen // tps
    kc = 128
    mesh = plsc.VectorSubcoreMesh(core_axis_name="c", subcore_axis_name="s")

    @functools.partial(
        pl.kernel,
        mesh=mesh,
        out_type=jax.ShapeDtypeStruct((tot, width), jnp.float32),
        scratch_types=[
            pltpu.VMEM((kc,), jnp.int32),
            pltpu.VMEM((kc, width), jnp.float32),
            pltpu.SemaphoreType.DMA,
        ],
    )
    def scatter_k(qv_hbm, grank_hbm, sqv_hbm, idx_v, row_v, sem):
        wid = lax.axis_index("s") * info.num_cores + lax.axis_index("c")
        seg = wid // tps
        hlf = wid % tps
        bidx = seg // _N_HASHES

        def chunk(ci, _):
            t0 = hlf * half + ci * kc
            pltpu.sync_copy(grank_hbm.at[seg, pl.ds(t0, kc)], idx_v)
            pltpu.sync_copy(qv_hbm.at[pl.ds(bidx * seqlen + t0, kc)], row_v)
            pltpu.async_copy(row_v, sqv_hbm.at[idx_v], sem).wait()
            return ()

        lax.fori_loop(0, half // kc, chunk, (), unroll=False)

    return scatter_k(qv_flat, grank)


def _attn_body(qv_ref, qvp_ref, stf_ref, stl_ref, so_ref, *, nchunk):
    ch = qv_ref.shape[2]
    dim = qv_ref.shape[3] // 2
    bb = pl.program_id(0)
    c = pl.program_id(1)
    segs_per_batch = _N_HASHES
    c0 = c * _CB
    nchunk_per_seg = nchunk // segs_per_batch
    s = c0 // nchunk_per_seg
    p = bb * segs_per_batch + s
    pprev = bb * segs_per_batch + (s + segs_per_batch - 1) % segs_per_batch
    is_boundary = (c0 % nchunk_per_seg) == 0
    qcol = stf_ref[pl.ds(p, 1)][0]
    kvrow = stl_ref[pl.ds(pprev, 1)][0]
    eqm = qcol == kvrow
    diag = (lax.broadcasted_iota(jnp.int32, (ch, ch), 0)
            == lax.broadcasted_iota(jnp.int32, (ch, ch), 1))
    for j in range(_CB):
        cur = qv_ref[0, j]
        prv = qv_ref[0, j - 1] if j > 0 else qvp_ref[0, 0]
        q = cur[:, :dim]
        kcat = jnp.concatenate([q, prv[:, :dim]], axis=0)
        norm = jnp.sqrt(jnp.sum(kcat * kcat, axis=1, keepdims=True))
        k = kcat / jnp.maximum(norm, 1e-12)
        dots = lax.dot_general(q, k, (((1,), (1,)), ((), ())),
                               preferred_element_type=jnp.float32) * 0.125
        d1 = jnp.where(diag, _SELF_VAL, dots[:, :ch])
        if j == 0:
            lb_mask = jnp.logical_and(is_boundary, eqm)
            d2 = jnp.where(lb_mask, _SELF_VAL, dots[:, ch:])
        else:
            d2 = dots[:, ch:]
        dots = jnp.concatenate([d1, d2], axis=1)
        m = jnp.max(dots, axis=1, keepdims=True)
        e = jnp.exp(dots - m)
        s_ = jnp.sum(e, axis=1, keepdims=True)
        lse = m + jnp.log(s_)
        probs = e / s_
        vv = jnp.concatenate([cur[:, dim:], prv[:, dim:]], axis=0)
        o = jnp.dot(probs, vv, preferred_element_type=jnp.float32)
        so_ref[0, j] = jnp.concatenate(
            [o, lse, jnp.zeros((ch, dim - 1), jnp.float32)], axis=1)


def _attention(sqv4, stf, stl):
    b, nchunk, ch, width = sqv4.shape
    ncb = nchunk // _CB
    nprog = stf.shape[0]

    def prev_map(bb, c):
        return (bb, (c * _CB - 1) % nchunk, 0, 0)

    call = pl.pallas_call(
        functools.partial(_attn_body, nchunk=nchunk),
        grid=(b, ncb),
        in_specs=[
            pl.BlockSpec((1, _CB, ch, width), lambda bb, c: (bb, c, 0, 0)),
            pl.BlockSpec((1, 1, ch, width), prev_map),
            pl.BlockSpec((nprog, ch, 1), lambda bb, c: (0, 0, 0)),
            pl.BlockSpec((nprog, 1, ch), lambda bb, c: (0, 0, 0)),
        ],
        out_specs=pl.BlockSpec((1, _CB, ch, width),
                               lambda bb, c: (bb, c, 0, 0)),
        out_shape=jax.ShapeDtypeStruct((b, nchunk, ch, width), jnp.float32),
    )
    return call(sqv4, sqv4, stf, stl)


def _sc_gather(so_flat, grank):
    nseg, seqlen = grank.shape
    tot = nseg * seqlen
    width = so_flat.shape[1]
    info = plsc.get_sparse_core_info()
    nw = info.num_cores * info.num_subcores
    per_w = tot // nw
    kc = 128
    mesh = plsc.VectorSubcoreMesh(core_axis_name="c", subcore_axis_name="s")
    grank_flat = grank.reshape(tot)

    @functools.partial(
        pl.kernel,
        mesh=mesh,
        out_type=jax.ShapeDtypeStruct((tot, width), jnp.float32),
        scratch_types=[
            pltpu.VMEM((kc,), jnp.int32),
            pltpu.VMEM((kc, width), jnp.float32),
            pltpu.SemaphoreType.DMA,
        ],
    )
    def gather_k(so_hbm, grank_hbm, og_hbm, idx_v, row_v, sem):
        wid = lax.axis_index("s") * info.num_cores + lax.axis_index("c")
        base = wid * per_w

        def chunk(ci, _):
            pltpu.sync_copy(grank_hbm.at[pl.ds(base + ci * kc, kc)], idx_v)
            pltpu.async_copy(so_hbm.at[idx_v], row_v, sem).wait()
            pltpu.sync_copy(row_v, og_hbm.at[pl.ds(base + ci * kc, kc)])
            return ()

        lax.fori_loop(0, per_w // kc, chunk, (), unroll=False)

    return gather_k(so_flat, grank_flat)


def _combine_body(og_ref, out_ref):
    dim = og_ref.shape[3] // 2
    ls = [og_ref[0, h][:, dim].reshape(1, -1) for h in range(_N_HASHES)]
    l = jnp.concatenate(ls, axis=0)
    m = jnp.max(l, axis=0, keepdims=True)
    w = jnp.exp(l - m)
    s = jnp.sum(w, axis=0, keepdims=True)
    p = w / s
    acc = og_ref[0, 0][:, :dim] * p[0][:, None]
    for h in range(1, _N_HASHES):
        acc = acc + og_ref[0, h][:, :dim] * p[h][:, None]
    out_ref[0] = acc


def _combine(og):
    b, nh, seqlen, width = og.shape
    dim = width // 2
    tb = 512
    nt = seqlen // tb
    return pl.pallas_call(
        _combine_body,
        grid=(b, nt),
        in_specs=[
            pl.BlockSpec((1, nh, tb, width), lambda bb, t: (bb, 0, t, 0)),
        ],
        out_specs=pl.BlockSpec((1, tb, dim), lambda bb, t: (bb, t, 0)),
        out_shape=jax.ShapeDtypeStruct((b, seqlen, dim), jnp.float32),
    )(og)


def kernel(vecs, v, rotations):
    b, seqlen, dim = vecs.shape
    rot_t = rotations[0].transpose(1, 0, 2)

    grank_f, stf, stl = _hash_rank(vecs, rot_t)
    grank = grank_f.astype(jnp.int32)

    qv_flat = jnp.concatenate([vecs, v], axis=-1).reshape(b * seqlen, 2 * dim)
    sqv_flat = _sc_scatter(qv_flat, grank)

    nchunk = _N_HASHES * _N_BUCKETS
    sqv4 = sqv_flat.reshape(b, nchunk, _CHUNK, 2 * dim)

    so = _attention(sqv4, stf, stl)

    so_flat = so.reshape(b * nchunk * _CHUNK, 2 * dim)
    og_flat = _sc_gather(so_flat, grank)

    og = og_flat.reshape(b, _N_HASHES, seqlen, 2 * dim)
    return _combine(og)


# speedup vs baseline: 6.0883x; 1.7486x over previous
import functools

import jax
import jax.numpy as jnp
from jax import lax
from jax.experimental import pallas as pl
from jax.experimental.pallas import tpu as pltpu
from jax.experimental.pallas import tpu_sc as plsc

_N_BUCKETS = 128
_N_HASHES = 4
_SELF_VAL = -50000.0
_CHUNK = 64
_CB = 32
_RBLK = 512


def _rank_body(vecs_ref, rot_ref, rank_ref, stf_ref, stl_ref, o_scr):
    p = pl.program_id(0)
    seqlen = vecs_ref.shape[1]
    nb = seqlen // _RBLK
    rot = rot_ref[0]

    iota_c = lax.broadcasted_iota(jnp.int32, (_RBLK, _N_BUCKETS), 1)
    lt = (lax.broadcasted_iota(jnp.int32, (_RBLK, _RBLK), 0)
          > lax.broadcasted_iota(jnp.int32, (_RBLK, _RBLK), 1)
          ).astype(jnp.bfloat16)

    wr_list = []
    hb_list = []
    for i in range(nb):
        blk = vecs_ref[0, pl.ds(i * _RBLK, _RBLK), :]
        d = jnp.dot(blk, rot, preferred_element_type=jnp.float32)
        full = jnp.concatenate([d, -d], axis=1)
        m = jnp.max(full, axis=1, keepdims=True)
        cand = jnp.where(full == m, iota_c, _N_BUCKETS)
        cstar = jnp.min(cand, axis=1, keepdims=True)
        onehot = (iota_c == cstar).astype(jnp.float32)
        o_scr[pl.ds(i * _RBLK, _RBLK), :] = onehot
        c = jnp.dot(lt, onehot.astype(jnp.bfloat16),
                    preferred_element_type=jnp.float32)
        wr_list.append(jnp.sum(c * onehot, axis=1))
        hb_list.append(jnp.sum(onehot, axis=0, keepdims=True))

    hb = jnp.concatenate(hb_list, axis=0)
    ltb = (lax.broadcasted_iota(jnp.int32, (nb, nb), 0)
           > lax.broadcasted_iota(jnp.int32, (nb, nb), 1)).astype(jnp.float32)
    boff = jnp.dot(ltb, hb, preferred_element_type=jnp.float32)
    htot = jnp.sum(hb, axis=0, keepdims=True)
    su = (lax.broadcasted_iota(jnp.int32, (_N_BUCKETS, _N_BUCKETS), 0)
          < lax.broadcasted_iota(jnp.int32, (_N_BUCKETS, _N_BUCKETS), 1)
          ).astype(jnp.float32)
    off = jnp.dot(htot, su, preferred_element_type=jnp.float32)
    base = off + boff

    gbase = (p * seqlen).astype(jnp.float32)
    iota64r = lax.broadcasted_iota(jnp.int32, (1, _CHUNK), 1).astype(jnp.float32)
    stf_acc = jnp.zeros((_CHUNK, 1), jnp.float32)
    stl_acc = jnp.zeros((1, _CHUNK), jnp.float32)
    for i in range(nb):
        onehot = o_scr[pl.ds(i * _RBLK, _RBLK), :]
        r = wr_list[i] + jnp.sum(onehot * base[i:i + 1, :], axis=1)
        rank_ref[0, 0, pl.ds(i * _RBLK, _RBLK)] = r + gbase
        rcol = r[:, None]
        tok_col = (lax.broadcasted_iota(jnp.int32, (_RBLK, 1), 0)
                   .astype(jnp.float32) + (i * _RBLK))
        m1 = (rcol == iota64r).astype(jnp.float32)
        m2 = (rcol == iota64r + (seqlen - _CHUNK)).astype(jnp.float32)
        stf_acc = stf_acc + lax.dot_general(
            m1, tok_col, (((0,), (0,)), ((), ())),
            preferred_element_type=jnp.float32)
        stl_acc = stl_acc + lax.dot_general(
            tok_col, m2, (((0,), (0,)), ((), ())),
            preferred_element_type=jnp.float32)
    stf_ref[0] = stf_acc
    stl_ref[0] = stl_acc


def _hash_rank(vecs, rot_t):
    b, seqlen, dim = vecs.shape
    nprog = b * _N_HASHES
    rank, stf, stl = pl.pallas_call(
        _rank_body,
        grid=(nprog,),
        in_specs=[
            pl.BlockSpec((1, seqlen, dim), lambda p: (p // _N_HASHES, 0, 0)),
            pl.BlockSpec((1, dim, dim), lambda p: (p % _N_HASHES, 0, 0)),
        ],
        out_specs=[
            pl.BlockSpec((1, 1, seqlen), lambda p: (p, 0, 0)),
            pl.BlockSpec((1, _CHUNK, 1), lambda p: (p, 0, 0)),
            pl.BlockSpec((1, 1, _CHUNK), lambda p: (p, 0, 0)),
        ],
        out_shape=[
            jax.ShapeDtypeStruct((nprog, 1, seqlen), jnp.float32),
            jax.ShapeDtypeStruct((nprog, _CHUNK, 1), jnp.float32),
            jax.ShapeDtypeStruct((nprog, 1, _CHUNK), jnp.float32),
        ],
        scratch_shapes=[pltpu.VMEM((seqlen, _N_BUCKETS), jnp.float32)],
    )(vecs, rot_t)
    return rank.reshape(nprog, seqlen), stf, stl


def _sc_scatter(qv_flat, grank):
    nseg, seqlen = grank.shape
    tot = nseg * seqlen
    width = qv_flat.shape[1]
    info = plsc.get_sparse_core_info()
    nw = info.num_cores * info.num_subcores
    tps = nw // nseg
    half = seqlen // tps
    kc = 128
    mesh = plsc.VectorSubcoreMesh(core_axis_name="c", subcore_axis_name="s")

    @functools.partial(
        pl.kernel,
        mesh=mesh,
        out_type=jax.ShapeDtypeStruct((tot, width), jnp.float32),
        scratch_types=[
            pltpu.VMEM((kc,), jnp.int32),
            pltpu.VMEM((kc, width), jnp.float32),
            pltpu.SemaphoreType.DMA,
        ],
    )
    def scatter_k(qv_hbm, grank_hbm, sqv_hbm, idx_v, row_v, sem):
        wid = lax.axis_index("s") * info.num_cores + lax.axis_index("c")
        seg = wid // tps
        hlf = wid % tps
        bidx = seg // _N_HASHES

        def chunk(ci, _):
            t0 = hlf * half + ci * kc
            pltpu.sync_copy(grank_hbm.at[seg, pl.ds(t0, kc)], idx_v)
            pltpu.sync_copy(qv_hbm.at[pl.ds(bidx * seqlen + t0, kc)], row_v)
            pltpu.async_copy(row_v, sqv_hbm.at[idx_v], sem).wait()
            return ()

        lax.fori_loop(0, half // kc, chunk, (), unroll=False)

    return scatter_k(qv_flat, grank)


def _attn_body(qv_ref, qvp_ref, stf_ref, stl_ref, so_ref, *, nchunk):
    ch = qv_ref.shape[2]
    dim = qv_ref.shape[3] // 2
    bb = pl.program_id(0)
    c = pl.program_id(1)
    segs_per_batch = _N_HASHES
    c0 = c * _CB
    nchunk_per_seg = nchunk // segs_per_batch
    s = c0 // nchunk_per_seg
    p = bb * segs_per_batch + s
    pprev = bb * segs_per_batch + (s + segs_per_batch - 1) % segs_per_batch
    is_boundary = (c0 % nchunk_per_seg) == 0
    qcol = stf_ref[pl.ds(p, 1)][0]
    kvrow = stl_ref[pl.ds(pprev, 1)][0]
    diag_add = jnp.where(
        lax.broadcasted_iota(jnp.int32, (ch, ch), 0)
        == lax.broadcasted_iota(jnp.int32, (ch, ch), 1),
        jnp.float32(-1e4), jnp.float32(0.0))
    bnd_add = jnp.where(
        jnp.logical_and(is_boundary, qcol == kvrow),
        jnp.float32(-1e4), jnp.float32(0.0))
    zpad = jnp.zeros((ch, dim - 1), jnp.float32)
    madd0 = jnp.concatenate([diag_add, bnd_add], axis=1)
    madd_rest = jnp.concatenate(
        [diag_add, jnp.zeros((ch, ch), jnp.float32)], axis=1)

    kn_prev = None
    vv_prev = None
    for j in range(_CB):
        cur = qv_ref[0, j]
        q = cur[:, :dim]
        norm = jnp.sqrt(jnp.sum(q * q, axis=1, keepdims=True))
        kn = q / jnp.maximum(norm, 1e-12)
        vcur = cur[:, dim:]
        if j == 0:
            prv = qvp_ref[0, 0]
            qp = prv[:, :dim]
            pnorm = jnp.sqrt(jnp.sum(qp * qp, axis=1, keepdims=True))
            kn_prev = qp / jnp.maximum(pnorm, 1e-12)
            vv_prev = prv[:, dim:]
        kcat = jnp.concatenate([kn, kn_prev], axis=0)
        vcat = jnp.concatenate([vcur, vv_prev], axis=0)
        q8 = q * 0.125
        dots = lax.dot_general(q8, kcat, (((1,), (1,)), ((), ())),
                               preferred_element_type=jnp.float32)
        e = jnp.exp(dots + (madd0 if j == 0 else madd_rest))
        s_ = jnp.sum(e, axis=1, keepdims=True)
        lse = jnp.log(s_)
        o = jnp.dot(e, vcat, preferred_element_type=jnp.float32) / s_
        so_ref[0, j] = jnp.concatenate([o, lse, zpad], axis=1)
        kn_prev = kn
        vv_prev = vcur


def _attention(sqv4, stf, stl):
    b, nchunk, ch, width = sqv4.shape
    ncb = nchunk // _CB
    nprog = stf.shape[0]

    def prev_map(bb, c):
        return (bb, (c * _CB - 1) % nchunk, 0, 0)

    call = pl.pallas_call(
        functools.partial(_attn_body, nchunk=nchunk),
        grid=(b, ncb),
        in_specs=[
            pl.BlockSpec((1, _CB, ch, width), lambda bb, c: (bb, c, 0, 0)),
            pl.BlockSpec((1, 1, ch, width), prev_map),
            pl.BlockSpec((nprog, ch, 1), lambda bb, c: (0, 0, 0)),
            pl.BlockSpec((nprog, 1, ch), lambda bb, c: (0, 0, 0)),
        ],
        out_specs=pl.BlockSpec((1, _CB, ch, width),
                               lambda bb, c: (bb, c, 0, 0)),
        out_shape=jax.ShapeDtypeStruct((b, nchunk, ch, width), jnp.float32),
    )
    return call(sqv4, sqv4, stf, stl)


def _sc_gather(so_flat, grank):
    nseg, seqlen = grank.shape
    tot = nseg * seqlen
    width = so_flat.shape[1]
    info = plsc.get_sparse_core_info()
    nw = info.num_cores * info.num_subcores
    per_w = tot // nw
    kc = 128
    mesh = plsc.VectorSubcoreMesh(core_axis_name="c", subcore_axis_name="s")
    grank_flat = grank.reshape(tot)

    @functools.partial(
        pl.kernel,
        mesh=mesh,
        out_type=jax.ShapeDtypeStruct((tot, width), jnp.float32),
        scratch_types=[
            pltpu.VMEM((kc,), jnp.int32),
            pltpu.VMEM((kc, width), jnp.float32),
            pltpu.SemaphoreType.DMA,
        ],
    )
    def gather_k(so_hbm, grank_hbm, og_hbm, idx_v, row_v, sem):
        wid = lax.axis_index("s") * info.num_cores + lax.axis_index("c")
        base = wid * per_w

        def chunk(ci, _):
            pltpu.sync_copy(grank_hbm.at[pl.ds(base + ci * kc, kc)], idx_v)
            pltpu.async_copy(so_hbm.at[idx_v], row_v, sem).wait()
            pltpu.sync_copy(row_v, og_hbm.at[pl.ds(base + ci * kc, kc)])
            return ()

        lax.fori_loop(0, per_w // kc, chunk, (), unroll=False)

    return gather_k(so_flat, grank_flat)


def _combine_body(og_ref, out_ref):
    dim = og_ref.shape[3] // 2
    ls = [og_ref[0, h][:, dim].reshape(1, -1) for h in range(_N_HASHES)]
    l = jnp.concatenate(ls, axis=0)
    m = jnp.max(l, axis=0, keepdims=True)
    w = jnp.exp(l - m)
    s = jnp.sum(w, axis=0, keepdims=True)
    p = w / s
    acc = og_ref[0, 0][:, :dim] * p[0][:, None]
    for h in range(1, _N_HASHES):
        acc = acc + og_ref[0, h][:, :dim] * p[h][:, None]
    out_ref[0] = acc


def _combine(og):
    b, nh, seqlen, width = og.shape
    dim = width // 2
    tb = 512
    nt = seqlen // tb
    return pl.pallas_call(
        _combine_body,
        grid=(b, nt),
        in_specs=[
            pl.BlockSpec((1, nh, tb, width), lambda bb, t: (bb, 0, t, 0)),
        ],
        out_specs=pl.BlockSpec((1, tb, dim), lambda bb, t: (bb, t, 0)),
        out_shape=jax.ShapeDtypeStruct((b, seqlen, dim), jnp.float32),
    )(og)


def kernel(vecs, v, rotations):
    b, seqlen, dim = vecs.shape
    rot_t = rotations[0].transpose(1, 0, 2)

    grank_f, stf, stl = _hash_rank(vecs, rot_t)
    grank = grank_f.astype(jnp.int32)

    qv_flat = jnp.concatenate([vecs, v], axis=-1).reshape(b * seqlen, 2 * dim)
    sqv_flat = _sc_scatter(qv_flat, grank)

    nchunk = _N_HASHES * _N_BUCKETS
    sqv4 = sqv_flat.reshape(b, nchunk, _CHUNK, 2 * dim)

    so = _attention(sqv4, stf, stl)

    so_flat = so.reshape(b * nchunk * _CHUNK, 2 * dim)
    og_flat = _sc_gather(so_flat, grank)

    og = og_flat.reshape(b, _N_HASHES, seqlen, 2 * dim)
    return _combine(og)


# speedup vs baseline: 6.5590x; 1.0773x over previous
import functools

import jax
import jax.numpy as jnp
from jax import lax
from jax.experimental import pallas as pl
from jax.experimental.pallas import tpu as pltpu
from jax.experimental.pallas import tpu_sc as plsc

_N_BUCKETS = 128
_N_HASHES = 4
_SELF_VAL = -50000.0
_CHUNK = 64
_CB = 32
_RBLK = 512


def _rank_body(vecs_ref, rot_ref, rank_ref, stf_ref, stl_ref, o_scr):
    p = pl.program_id(0)
    seqlen = vecs_ref.shape[1]
    nb = seqlen // _RBLK
    rot = rot_ref[0]

    iota_cf = lax.broadcasted_iota(
        jnp.int32, (_RBLK, _N_BUCKETS), 1).astype(jnp.float32)
    lt = (lax.broadcasted_iota(jnp.int32, (_RBLK, _RBLK), 0)
          > lax.broadcasted_iota(jnp.int32, (_RBLK, _RBLK), 1)
          ).astype(jnp.bfloat16)

    wr_list = []
    hb_list = []
    for i in range(nb):
        blk = vecs_ref[0, pl.ds(i * _RBLK, _RBLK), :]
        d = jnp.dot(blk, rot, preferred_element_type=jnp.float32)
        full = jnp.concatenate([d, -d], axis=1)
        m = jnp.max(full, axis=1, keepdims=True)
        cand = jnp.where(full == m, iota_cf, jnp.float32(_N_BUCKETS))
        cstar = jnp.min(cand, axis=1, keepdims=True)
        onehot = (iota_cf == cstar).astype(jnp.float32)
        o_scr[pl.ds(i * _RBLK, _RBLK), :] = onehot
        c = jnp.dot(lt, onehot.astype(jnp.bfloat16),
                    preferred_element_type=jnp.float32)
        wr_list.append(jnp.sum(c * onehot, axis=1))
        hb_list.append(jnp.sum(onehot, axis=0, keepdims=True))

    hb = jnp.concatenate(hb_list, axis=0)
    ltb = (lax.broadcasted_iota(jnp.int32, (nb, nb), 0)
           > lax.broadcasted_iota(jnp.int32, (nb, nb), 1)).astype(jnp.float32)
    boff = jnp.dot(ltb, hb, preferred_element_type=jnp.float32)
    htot = jnp.sum(hb, axis=0, keepdims=True)
    su = (lax.broadcasted_iota(jnp.int32, (_N_BUCKETS, _N_BUCKETS), 0)
          < lax.broadcasted_iota(jnp.int32, (_N_BUCKETS, _N_BUCKETS), 1)
          ).astype(jnp.float32)
    off = jnp.dot(htot, su, preferred_element_type=jnp.float32)
    base = off + boff

    gbase = (p * seqlen).astype(jnp.float32)
    iota64r = lax.broadcasted_iota(jnp.int32, (1, _CHUNK), 1).astype(jnp.float32)
    stf_acc = jnp.zeros((_CHUNK, 1), jnp.float32)
    stl_acc = jnp.zeros((1, _CHUNK), jnp.float32)
    for i in range(nb):
        onehot = o_scr[pl.ds(i * _RBLK, _RBLK), :]
        r = wr_list[i] + jnp.sum(onehot * base[i:i + 1, :], axis=1)
        rank_ref[0, pl.ds(i * _RBLK, _RBLK), 0] = r + gbase
        rcol = r[:, None]
        tok_col = (lax.broadcasted_iota(jnp.int32, (_RBLK, 1), 0)
                   .astype(jnp.float32) + (i * _RBLK))
        m1 = (rcol == iota64r).astype(jnp.float32)
        m2 = (rcol == iota64r + (seqlen - _CHUNK)).astype(jnp.float32)
        stf_acc = stf_acc + lax.dot_general(
            m1, tok_col, (((0,), (0,)), ((), ())),
            preferred_element_type=jnp.float32)
        stl_acc = stl_acc + lax.dot_general(
            tok_col, m2, (((0,), (0,)), ((), ())),
            preferred_element_type=jnp.float32)
    stf_ref[0] = stf_acc
    stl_ref[0] = stl_acc


def _hash_rank(vecs, rot_t):
    b, seqlen, dim = vecs.shape
    nprog = b * _N_HASHES
    rank, stf, stl = pl.pallas_call(
        _rank_body,
        grid=(nprog,),
        in_specs=[
            pl.BlockSpec((1, seqlen, dim), lambda p: (p // _N_HASHES, 0, 0)),
            pl.BlockSpec((1, dim, dim), lambda p: (p % _N_HASHES, 0, 0)),
        ],
        out_specs=[
            pl.BlockSpec((1, seqlen, 1), lambda p: (p, 0, 0)),
            pl.BlockSpec((1, _CHUNK, 1), lambda p: (p, 0, 0)),
            pl.BlockSpec((1, 1, _CHUNK), lambda p: (p, 0, 0)),
        ],
        out_shape=[
            jax.ShapeDtypeStruct((nprog, seqlen, 1), jnp.float32),
            jax.ShapeDtypeStruct((nprog, _CHUNK, 1), jnp.float32),
            jax.ShapeDtypeStruct((nprog, 1, _CHUNK), jnp.float32),
        ],
        scratch_shapes=[pltpu.VMEM((seqlen, _N_BUCKETS), jnp.float32)],
    )(vecs, rot_t)
    return rank.reshape(nprog, seqlen), stf, stl


_NBUF = 3


def _sc_scatter(qv_flat, grank3):
    nseg, nrow, kc = grank3.shape
    seqlen = nrow * kc
    tot = nseg * seqlen
    width = qv_flat.shape[1]
    info = plsc.get_sparse_core_info()
    nw = info.num_cores * info.num_subcores
    tps = nw // nseg
    hrow = nrow // tps
    mesh = plsc.VectorSubcoreMesh(core_axis_name="c", subcore_axis_name="s")

    @functools.partial(
        pl.kernel,
        mesh=mesh,
        out_type=jax.ShapeDtypeStruct((tot, width), jnp.float32),
        scratch_types=[
            pltpu.VMEM((hrow, kc), jnp.int32),
            [pltpu.VMEM((kc, width), jnp.float32)] * _NBUF,
            [pltpu.SemaphoreType.DMA] * _NBUF,
            [pltpu.SemaphoreType.DMA] * _NBUF,
        ],
    )
    def scatter_k(qv_hbm, grank_hbm, sqv_hbm, idx_v, rows, semL, semS):
        wid = lax.axis_index("s") * info.num_cores + lax.axis_index("c")
        seg = wid // tps
        hlf = wid % tps
        bidx = seg // _N_HASHES
        row0 = bidx * seqlen + hlf * (hrow * kc)
        pltpu.sync_copy(grank_hbm.at[seg, pl.ds(hlf * hrow, hrow)], idx_v)

        cpl = [None] * hrow
        cps = [None] * hrow
        for ci in range(hrow):
            b = ci % _NBUF
            if ci >= _NBUF:
                cps[ci - _NBUF].wait()
            cpl[ci] = pltpu.async_copy(
                qv_hbm.at[pl.ds(row0 + ci * kc, kc)], rows[b], semL[b])
            if ci >= 1:
                pb = (ci - 1) % _NBUF
                cpl[ci - 1].wait()
                cps[ci - 1] = pltpu.async_copy(
                    rows[pb], sqv_hbm.at[idx_v.at[ci - 1]], semS[pb])
        cpl[hrow - 1].wait()
        cps[hrow - 1] = pltpu.async_copy(
            rows[(hrow - 1) % _NBUF],
            sqv_hbm.at[idx_v.at[hrow - 1]], semS[(hrow - 1) % _NBUF])
        for ci in range(hrow - _NBUF, hrow):
            cps[ci].wait()

    return scatter_k(qv_flat, grank3)


def _attn_body(qv_ref, qvp_ref, stf_ref, stl_ref, so_ref, *, nchunk):
    ch = qv_ref.shape[2]
    dim = qv_ref.shape[3] // 2
    bb = pl.program_id(0)
    c = pl.program_id(1)
    segs_per_batch = _N_HASHES
    c0 = c * _CB
    nchunk_per_seg = nchunk // segs_per_batch
    s = c0 // nchunk_per_seg
    p = bb * segs_per_batch + s
    pprev = bb * segs_per_batch + (s + segs_per_batch - 1) % segs_per_batch
    is_boundary = (c0 % nchunk_per_seg) == 0
    qcol = stf_ref[pl.ds(p, 1)][0]
    kvrow = stl_ref[pl.ds(pprev, 1)][0]
    diag_add = jnp.where(
        lax.broadcasted_iota(jnp.int32, (ch, ch), 0)
        == lax.broadcasted_iota(jnp.int32, (ch, ch), 1),
        jnp.float32(-1e4), jnp.float32(0.0))
    bnd_add = jnp.where(
        jnp.logical_and(is_boundary, qcol == kvrow),
        jnp.float32(-1e4), jnp.float32(0.0))
    zpad = jnp.zeros((ch, dim - 1), jnp.float32)
    madd0 = jnp.concatenate([diag_add, bnd_add], axis=1)
    madd_rest = jnp.concatenate(
        [diag_add, jnp.zeros((ch, ch), jnp.float32)], axis=1)

    kn_prev = None
    vv_prev = None
    for j in range(_CB):
        cur = qv_ref[0, j]
        q = cur[:, :dim]
        n2 = jnp.sum(q * q, axis=1, keepdims=True)
        kn = q * lax.rsqrt(jnp.maximum(n2, 1e-24))
        vcur = cur[:, dim:]
        if j == 0:
            prv = qvp_ref[0, 0]
            qp = prv[:, :dim]
            pn2 = jnp.sum(qp * qp, axis=1, keepdims=True)
            kn_prev = qp * lax.rsqrt(jnp.maximum(pn2, 1e-24))
            vv_prev = prv[:, dim:]
        kcat = jnp.concatenate([kn, kn_prev], axis=0)
        vcat = jnp.concatenate([vcur, vv_prev], axis=0)
        q8 = q * 0.125
        dots = lax.dot_general(q8, kcat, (((1,), (1,)), ((), ())),
                               preferred_element_type=jnp.float32)
        e = jnp.exp(dots + (madd0 if j == 0 else madd_rest))
        s_ = jnp.sum(e, axis=1, keepdims=True)
        lse = jnp.log(s_)
        o = jnp.dot(e, vcat, preferred_element_type=jnp.float32) / s_
        so_ref[0, j] = jnp.concatenate([o, lse, zpad], axis=1)
        kn_prev = kn
        vv_prev = vcur


def _attention(sqv4, stf, stl):
    b, nchunk, ch, width = sqv4.shape
    ncb = nchunk // _CB
    nprog = stf.shape[0]

    def prev_map(bb, c):
        return (bb, (c * _CB - 1) % nchunk, 0, 0)

    call = pl.pallas_call(
        functools.partial(_attn_body, nchunk=nchunk),
        grid=(b, ncb),
        in_specs=[
            pl.BlockSpec((1, _CB, ch, width), lambda bb, c: (bb, c, 0, 0)),
            pl.BlockSpec((1, 1, ch, width), prev_map),
            pl.BlockSpec((nprog, ch, 1), lambda bb, c: (0, 0, 0)),
            pl.BlockSpec((nprog, 1, ch), lambda bb, c: (0, 0, 0)),
        ],
        out_specs=pl.BlockSpec((1, _CB, ch, width),
                               lambda bb, c: (bb, c, 0, 0)),
        out_shape=jax.ShapeDtypeStruct((b, nchunk, ch, width), jnp.float32),
    )
    return call(sqv4, sqv4, stf, stl)


def _sc_gather(so_flat, grank3):
    nseg, nrow, kc = grank3.shape
    seqlen = nrow * kc
    tot = nseg * seqlen
    width = so_flat.shape[1]
    info = plsc.get_sparse_core_info()
    nw = info.num_cores * info.num_subcores
    wrow = (nseg * nrow) // nw
    mesh = plsc.VectorSubcoreMesh(core_axis_name="c", subcore_axis_name="s")
    grank2 = grank3.reshape(nseg * nrow, kc)

    @functools.partial(
        pl.kernel,
        mesh=mesh,
        out_type=jax.ShapeDtypeStruct((tot, width), jnp.float32),
        scratch_types=[
            pltpu.VMEM((wrow, kc), jnp.int32),
            [pltpu.VMEM((kc, width), jnp.float32)] * _NBUF,
            [pltpu.SemaphoreType.DMA] * _NBUF,
            [pltpu.SemaphoreType.DMA] * _NBUF,
        ],
    )
    def gather_k(so_hbm, grank_hbm, og_hbm, idx_v, rows, semG, semW):
        wid = lax.axis_index("s") * info.num_cores + lax.axis_index("c")
        base = wid * (wrow * kc)
        pltpu.sync_copy(grank_hbm.at[pl.ds(wid * wrow, wrow)], idx_v)

        cpg = [None] * wrow
        cpw = [None] * wrow
        for ci in range(wrow):
            b = ci % _NBUF
            if ci >= _NBUF:
                cpw[ci - _NBUF].wait()
            cpg[ci] = pltpu.async_copy(
                so_hbm.at[idx_v.at[ci]], rows[b], semG[b])
            if ci >= 1:
                pb = (ci - 1) % _NBUF
                cpg[ci - 1].wait()
                cpw[ci - 1] = pltpu.async_copy(
                    rows[pb], og_hbm.at[pl.ds(base + (ci - 1) * kc, kc)],
                    semW[pb])
        cpg[wrow - 1].wait()
        cpw[wrow - 1] = pltpu.async_copy(
            rows[(wrow - 1) % _NBUF],
            og_hbm.at[pl.ds(base + (wrow - 1) * kc, kc)],
            semW[(wrow - 1) % _NBUF])
        for ci in range(wrow - _NBUF, wrow):
            cpw[ci].wait()

    return gather_k(so_flat, grank2)


def _combine_body(og_ref, out_ref):
    dim = og_ref.shape[3] // 2
    ls = [og_ref[0, h][:, dim].reshape(1, -1) for h in range(_N_HASHES)]
    l = jnp.concatenate(ls, axis=0)
    m = jnp.max(l, axis=0, keepdims=True)
    w = jnp.exp(l - m)
    s = jnp.sum(w, axis=0, keepdims=True)
    p = w / s
    acc = og_ref[0, 0][:, :dim] * p[0][:, None]
    for h in range(1, _N_HASHES):
        acc = acc + og_ref[0, h][:, :dim] * p[h][:, None]
    out_ref[0] = acc


def _combine(og):
    b, nh, seqlen, width = og.shape
    dim = width // 2
    tb = 512
    nt = seqlen // tb
    return pl.pallas_call(
        _combine_body,
        grid=(b, nt),
        in_specs=[
            pl.BlockSpec((1, nh, tb, width), lambda bb, t: (bb, 0, t, 0)),
        ],
        out_specs=pl.BlockSpec((1, tb, dim), lambda bb, t: (bb, t, 0)),
        out_shape=jax.ShapeDtypeStruct((b, seqlen, dim), jnp.float32),
    )(og)


def kernel(vecs, v, rotations):
    b, seqlen, dim = vecs.shape
    rot_t = rotations[0].transpose(1, 0, 2)

    grank_f, stf, stl = _hash_rank(vecs, rot_t)
    grank3 = grank_f.astype(jnp.int32).reshape(
        _N_HASHES * b, seqlen // 128, 128)

    qv_flat = jnp.concatenate([vecs, v], axis=-1).reshape(b * seqlen, 2 * dim)
    sqv_flat = _sc_scatter(qv_flat, grank3)

    nchunk = _N_HASHES * _N_BUCKETS
    sqv4 = sqv_flat.reshape(b, nchunk, _CHUNK, 2 * dim)

    so = _attention(sqv4, stf, stl)

    so_flat = so.reshape(b * nchunk * _CHUNK, 2 * dim)
    og_flat = _sc_gather(so_flat, grank3)

    og = og_flat.reshape(b, _N_HASHES, seqlen, 2 * dim)
    return _combine(og)
